# Initial kernel scaffold; baseline (speedup 1.0000x reference)
import jax
import jax.numpy as jnp
from jax.experimental import pallas as pl


def kernel(x, edge_index, W1l, W1r, b1, W2l, W2r, b2, W3l, W3r, b3, Wlin, blin):
    raise NotImplementedError("write your pallas kernel here")


# speedup vs baseline: 4.8248x; 4.8248x over previous
import functools

import jax
import jax.numpy as jnp
from jax import lax
from jax.experimental import pallas as pl
from jax.experimental.pallas import tpu as pltpu
from jax.experimental.pallas import tpu_sc as plsc

N = 10000
E = 320000
D = 128
NC = 2
NS = 16
NW = NC * NS
EPW = E // NW
CHUNK = 80
NCHUNK = EPW // CHUNK
RPT = 624
TAIL = N - RPT * NS
ZROWS = 24
CW = 16


def _fill2d(ref, nrows, ncols, value):
    vec = jnp.full((16,), value, jnp.float32)

    def row(r, _):
        def col(c, _2):
            ref[r, pl.ds(c * 16, 16)] = vec
            return 0
        return lax.fori_loop(0, ncols // 16, col, 0)
    lax.fori_loop(0, nrows, row, 0)


def _make_sc_agg(with_counts):
    mesh = plsc.VectorSubcoreMesh(core_axis_name="c", subcore_axis_name="s")
    out_type = [jax.ShapeDtypeStruct((NC, N, D), jnp.float32)]
    scratch = [
        pltpu.VMEM_SHARED((N, D), jnp.float32),
        pltpu.VMEM((ZROWS, D), jnp.float32),
        pltpu.VMEM((CHUNK,), jnp.int32),
        pltpu.VMEM((CHUNK,), jnp.int32),
        pltpu.VMEM((CHUNK, D), jnp.float32),
        pltpu.SemaphoreType.DMA,
    ]
    if with_counts:
        out_type.append(jax.ShapeDtypeStruct((NC, N, CW), jnp.float32))
        scratch += [
            pltpu.VMEM_SHARED((N, CW), jnp.float32),
            pltpu.VMEM((ZROWS, CW), jnp.float32),
            pltpu.VMEM((CHUNK, CW), jnp.float32),
        ]

    @functools.partial(pl.kernel, mesh=mesh, out_type=tuple(out_type),
                       scratch_types=tuple(scratch),
                       compiler_params=pltpu.CompilerParams(
                           use_tc_tiling_on_sc=False))
    def k(x_hbm, src_hbm, dst_hbm, *refs):
        if with_counts:
            (agg_out, cnt_out, agg_sh, zbuf, idx_s, idx_d, rows, sem,
             cnt_sh, zcnt, ones) = refs
        else:
            agg_out, agg_sh, zbuf, idx_s, idx_d, rows, sem = refs
        cid = lax.axis_index("c")
        sid = lax.axis_index("s")
        wid = cid * NS + sid

        is_last = sid == NS - 1

        _fill2d(zbuf, ZROWS, D, 0.0)
        for j in range(RPT // ZROWS):
            pltpu.sync_copy(zbuf, agg_sh.at[pl.ds(sid * RPT + j * ZROWS, ZROWS)])

        @pl.when(is_last)
        def _():
            pltpu.sync_copy(zbuf.at[pl.ds(0, TAIL)],
                            agg_sh.at[pl.ds(NS * RPT, TAIL)])
        if with_counts:
            _fill2d(zcnt, ZROWS, CW, 0.0)
            for j in range(RPT // ZROWS):
                pltpu.sync_copy(zcnt, cnt_sh.at[pl.ds(sid * RPT + j * ZROWS, ZROWS)])

            @pl.when(is_last)
            def _():
                pltpu.sync_copy(zcnt.at[pl.ds(0, TAIL)],
                                cnt_sh.at[pl.ds(NS * RPT, TAIL)])
            _fill2d(ones, CHUNK, CW, 1.0)
        plsc.subcore_barrier()

        base = wid * EPW

        def step(i, _):
            off = pl.multiple_of(base + i * CHUNK, 8)
            pltpu.sync_copy(src_hbm.at[pl.ds(off, CHUNK)], idx_s)
            pltpu.sync_copy(dst_hbm.at[pl.ds(off, CHUNK)], idx_d)
            pltpu.async_copy(x_hbm.at[idx_s], rows, sem).wait()
            pltpu.sync_copy(rows, agg_sh.at[idx_d], add=True)
            if with_counts:
                pltpu.sync_copy(ones, cnt_sh.at[idx_d], add=True)
            return 0
        lax.fori_loop(0, NCHUNK, step, 0)

        plsc.subcore_barrier()

        r0 = sid * RPT
        pltpu.sync_copy(agg_sh.at[pl.ds(r0, RPT)],
                        agg_out.at[cid, pl.ds(r0, RPT)])

        @pl.when(is_last)
        def _():
            pltpu.sync_copy(agg_sh.at[pl.ds(NS * RPT, TAIL)],
                            agg_out.at[cid, pl.ds(NS * RPT, TAIL)])
        if with_counts:
            pltpu.sync_copy(cnt_sh.at[pl.ds(r0, RPT)],
                            cnt_out.at[cid, pl.ds(r0, RPT)])

            @pl.when(is_last)
            def _():
                pltpu.sync_copy(cnt_sh.at[pl.ds(NS * RPT, TAIL)],
                                cnt_out.at[cid, pl.ds(NS * RPT, TAIL)])

    return k


_sc_agg_cnt = _make_sc_agg(True)
_sc_agg = _make_sc_agg(False)

_RB = 1000


def _tc_body(agg_ref, cnt_ref, x_ref, wl_ref, wr_ref, b_ref, o_ref):
    agg = agg_ref[0] + agg_ref[1]
    cnt = cnt_ref[0, :, 0:1] + cnt_ref[1, :, 0:1]
    m = agg / jnp.maximum(cnt, 1.0)
    h = lax.dot_general(m, wl_ref[...], (((1,), (1,)), ((), ())),
                        preferred_element_type=jnp.float32)
    h += lax.dot_general(x_ref[...], wr_ref[...], (((1,), (1,)), ((), ())),
                         preferred_element_type=jnp.float32)
    h += b_ref[...]
    o_ref[...] = jnp.maximum(h, 0.0)


def _tc_body_final(agg_ref, cnt_ref, x_ref, wl_ref, wr_ref, b_ref,
                   wlin_ref, blin_ref, o_ref):
    agg = agg_ref[0] + agg_ref[1]
    cnt = cnt_ref[0, :, 0:1] + cnt_ref[1, :, 0:1]
    m = agg / jnp.maximum(cnt, 1.0)
    h = lax.dot_general(m, wl_ref[...], (((1,), (1,)), ((), ())),
                        preferred_element_type=jnp.float32)
    h += lax.dot_general(x_ref[...], wr_ref[...], (((1,), (1,)), ((), ())),
                         preferred_element_type=jnp.float32)
    h += b_ref[...]
    h = jnp.maximum(h, 0.0)
    o_ref[...] = lax.dot_general(h, wlin_ref[...], (((1,), (1,)), ((), ())),
                                 preferred_element_type=jnp.float32) + blin_ref[...]


def _tc_layer(aggp, cntp, x, Wl, Wr, b, Wlin=None, blin=None):
    final = Wlin is not None
    grid = (N // _RB,)
    in_specs = [
        pl.BlockSpec((NC, _RB, D), lambda i: (0, i, 0)),
        pl.BlockSpec((NC, _RB, CW), lambda i: (0, i, 0)),
        pl.BlockSpec((_RB, D), lambda i: (i, 0)),
        pl.BlockSpec((D, D), lambda i: (0, 0)),
        pl.BlockSpec((D, D), lambda i: (0, 0)),
        pl.BlockSpec((1, D), lambda i: (0, 0)),
    ]
    args = [aggp, cntp, x, Wl, Wr, b.reshape(1, D)]
    if final:
        in_specs += [pl.BlockSpec((D, D), lambda i: (0, 0)),
                     pl.BlockSpec((1, D), lambda i: (0, 0))]
        args += [Wlin, blin.reshape(1, D)]
    return pl.pallas_call(
        _tc_body_final if final else _tc_body,
        grid=grid,
        in_specs=in_specs,
        out_specs=pl.BlockSpec((_RB, D), lambda i: (i, 0)),
        out_shape=jax.ShapeDtypeStruct((N, D), jnp.float32),
    )(*args)


def kernel(x, edge_index, W1l, W1r, b1, W2l, W2r, b2, W3l, W3r, b3,
           Wlin, blin):
    src = edge_index[0]
    dst = edge_index[1]
    aggp, cntp = _sc_agg_cnt(x, src, dst)
    h1 = _tc_layer(aggp, cntp, x, W1l, W1r, b1)
    (aggp2,) = _sc_agg(h1, src, dst)
    h2 = _tc_layer(aggp2, cntp, h1, W2l, W2r, b2)
    (aggp3,) = _sc_agg(h2, src, dst)
    out = _tc_layer(aggp3, cntp, h2, W3l, W3r, b3, Wlin, blin)
    return out


# speedup vs baseline: 8.9759x; 1.8604x over previous
import functools

import jax
import jax.numpy as jnp
from jax import lax
from jax.experimental import pallas as pl
from jax.experimental.pallas import tpu as pltpu
from jax.experimental.pallas import tpu_sc as plsc

N = 10000
E = 320000
D = 128
NC = 2
NS = 16
NW = NC * NS
EPW = E // NW
CHUNK = 80
NCHUNK = EPW // CHUNK
RPT = 624
TAIL = N - RPT * NS
ZROWS = 24
CW = 16


def _fill2d(ref, nrows, ncols, value):
    vec = jnp.full((16,), value, jnp.float32)

    def row(r, _):
        def col(c, _2):
            ref[r, pl.ds(c * 16, 16)] = vec
            return 0
        return lax.fori_loop(0, ncols // 16, col, 0)
    lax.fori_loop(0, nrows, row, 0)


def _make_sc_agg(with_counts):
    mesh = plsc.VectorSubcoreMesh(core_axis_name="c", subcore_axis_name="s")
    out_type = [jax.ShapeDtypeStruct((NC, N, D), jnp.float32)]
    scratch = [
        pltpu.VMEM_SHARED((N, D), jnp.float32),
        pltpu.VMEM((NCHUNK, CHUNK), jnp.int32),
        pltpu.VMEM((CHUNK,), jnp.int32),
        pltpu.VMEM((CHUNK,), jnp.int32),
        pltpu.VMEM((CHUNK, D), jnp.float32),
        pltpu.VMEM((CHUNK, D), jnp.float32),
        pltpu.SemaphoreType.DMA,
        pltpu.SemaphoreType.DMA,
        pltpu.SemaphoreType.DMA,
        pltpu.SemaphoreType.DMA,
        pltpu.SemaphoreType.DMA,
        pltpu.SemaphoreType.DMA,
    ]
    if with_counts:
        out_type.append(jax.ShapeDtypeStruct((NC, N, CW), jnp.float32))
        scratch += [
            pltpu.VMEM_SHARED((N, CW), jnp.float32),
            pltpu.VMEM((CHUNK, CW), jnp.float32),
            pltpu.SemaphoreType.DMA,
            pltpu.SemaphoreType.DMA,
        ]

    @functools.partial(pl.kernel, mesh=mesh, out_type=tuple(out_type),
                       scratch_types=tuple(scratch),
                       compiler_params=pltpu.CompilerParams(
                           use_tc_tiling_on_sc=False))
    def k(x_hbm, src_hbm, dst_hbm, *refs):
        if with_counts:
            (agg_out, cnt_out, agg_sh, didx, six0, six1, rows0, rows1,
             sg0, sg1, ss0, ss1, si0, si1, cnt_sh, ones, sc0, sc1) = refs
        else:
            (agg_out, agg_sh, didx, six0, six1, rows0, rows1,
             sg0, sg1, ss0, ss1, si0, si1) = refs
        six = (six0, six1)
        rows = (rows0, rows1)
        sg = (sg0, sg1)
        ss = (ss0, ss1)
        si = (si0, si1)
        if with_counts:
            sc = (sc0, sc1)
        cid = lax.axis_index("c")
        sid = lax.axis_index("s")
        wid = cid * NS + sid
        is_last = sid == NS - 1
        brow = wid * NCHUNK

        pltpu.sync_copy(dst_hbm.at[pl.ds(brow, NCHUNK)], didx)

        _fill2d(rows0, CHUNK, D, 0.0)
        for j in range(RPT // CHUNK):
            pltpu.sync_copy(rows0, agg_sh.at[pl.ds(sid * RPT + j * CHUNK, CHUNK)])
        rem = RPT - (RPT // CHUNK) * CHUNK
        pltpu.sync_copy(rows0.at[pl.ds(0, rem)],
                        agg_sh.at[pl.ds(sid * RPT + RPT - rem, rem)])

        @pl.when(is_last)
        def _():
            pltpu.sync_copy(rows0.at[pl.ds(0, TAIL)],
                            agg_sh.at[pl.ds(NS * RPT, TAIL)])
        if with_counts:
            _fill2d(ones, CHUNK, CW, 0.0)
            for j in range(RPT // CHUNK):
                pltpu.sync_copy(ones, cnt_sh.at[pl.ds(sid * RPT + j * CHUNK, CHUNK)])
            pltpu.sync_copy(ones.at[pl.ds(0, rem)],
                            cnt_sh.at[pl.ds(sid * RPT + RPT - rem, rem)])

            @pl.when(is_last)
            def _():
                pltpu.sync_copy(ones.at[pl.ds(0, TAIL)],
                                cnt_sh.at[pl.ds(NS * RPT, TAIL)])
            _fill2d(ones, CHUNK, CW, 1.0)
        plsc.subcore_barrier()

        pltpu.sync_copy(src_hbm.at[brow], six[0])
        pltpu.async_copy(x_hbm.at[six[0]], rows[0], sg[0])
        pltpu.async_copy(src_hbm.at[brow + 1], six[1], si[1])

        def chunk(i, _):
            b = lax.rem(i, 2)

            def piece(bs):
                o = 1 - bs
                pltpu.make_async_copy(x_hbm.at[six[bs]], rows[bs], sg[bs]).wait()
                nxt = jnp.minimum(i + 2, NCHUNK - 1)
                pltpu.async_copy(src_hbm.at[brow + nxt], six[bs], si[bs])
                pltpu.async_copy(rows[bs], agg_sh.at[didx.at[i]], ss[bs],
                                 add=True)
                if with_counts:
                    pltpu.async_copy(ones, cnt_sh.at[didx.at[i]], sc[bs],
                                     add=True)

                @pl.when(i > 0)
                def _():
                    pltpu.make_async_copy(rows[o], agg_sh.at[didx.at[i]],
                                          ss[o]).wait()
                    if with_counts:
                        pltpu.make_async_copy(ones, cnt_sh.at[didx.at[i]],
                                              sc[o]).wait()
                pltpu.make_async_copy(src_hbm.at[brow], six[o], si[o]).wait()
                pltpu.async_copy(x_hbm.at[six[o]], rows[o], sg[o])

            lax.cond(b == 0, lambda: piece(0), lambda: piece(1))
            return 0
        lax.fori_loop(0, NCHUNK - 1, chunk, 0)

        lb = (NCHUNK - 1) % 2
        lo = 1 - lb
        pltpu.make_async_copy(x_hbm.at[six[lb]], rows[lb], sg[lb]).wait()
        pltpu.async_copy(rows[lb], agg_sh.at[didx.at[NCHUNK - 1]], ss[lb],
                         add=True)
        if with_counts:
            pltpu.async_copy(ones, cnt_sh.at[didx.at[NCHUNK - 1]], sc[lb],
                             add=True)
        pltpu.make_async_copy(rows[lb], agg_sh.at[didx.at[0]], ss[lb]).wait()
        pltpu.make_async_copy(rows[lo], agg_sh.at[didx.at[0]], ss[lo]).wait()
        pltpu.make_async_copy(src_hbm.at[brow], six[lo], si[lo]).wait()
        if with_counts:
            pltpu.make_async_copy(ones, cnt_sh.at[didx.at[0]], sc[lb]).wait()
            pltpu.make_async_copy(ones, cnt_sh.at[didx.at[0]], sc[lo]).wait()

        plsc.subcore_barrier()

        r0 = sid * RPT
        pltpu.sync_copy(agg_sh.at[pl.ds(r0, RPT)],
                        agg_out.at[cid, pl.ds(r0, RPT)])

        @pl.when(is_last)
        def _():
            pltpu.sync_copy(agg_sh.at[pl.ds(NS * RPT, TAIL)],
                            agg_out.at[cid, pl.ds(NS * RPT, TAIL)])
        if with_counts:
            pltpu.sync_copy(cnt_sh.at[pl.ds(r0, RPT)],
                            cnt_out.at[cid, pl.ds(r0, RPT)])

            @pl.when(is_last)
            def _():
                pltpu.sync_copy(cnt_sh.at[pl.ds(NS * RPT, TAIL)],
                                cnt_out.at[cid, pl.ds(NS * RPT, TAIL)])

    return k


_sc_agg_cnt = _make_sc_agg(True)
_sc_agg = _make_sc_agg(False)

_RB = 1000


def _tc_body(agg_ref, cnt_ref, x_ref, wl_ref, wr_ref, b_ref, o_ref):
    agg = agg_ref[0] + agg_ref[1]
    cnt = cnt_ref[0, :, 0:1] + cnt_ref[1, :, 0:1]
    m = agg / jnp.maximum(cnt, 1.0)
    h = lax.dot_general(m, wl_ref[...], (((1,), (1,)), ((), ())),
                        preferred_element_type=jnp.float32)
    h += lax.dot_general(x_ref[...], wr_ref[...], (((1,), (1,)), ((), ())),
                         preferred_element_type=jnp.float32)
    h += b_ref[...]
    o_ref[...] = jnp.maximum(h, 0.0)


def _tc_body_final(agg_ref, cnt_ref, x_ref, wl_ref, wr_ref, b_ref,
                   wlin_ref, blin_ref, o_ref):
    agg = agg_ref[0] + agg_ref[1]
    cnt = cnt_ref[0, :, 0:1] + cnt_ref[1, :, 0:1]
    m = agg / jnp.maximum(cnt, 1.0)
    h = lax.dot_general(m, wl_ref[...], (((1,), (1,)), ((), ())),
                        preferred_element_type=jnp.float32)
    h += lax.dot_general(x_ref[...], wr_ref[...], (((1,), (1,)), ((), ())),
                         preferred_element_type=jnp.float32)
    h += b_ref[...]
    h = jnp.maximum(h, 0.0)
    o_ref[...] = lax.dot_general(h, wlin_ref[...], (((1,), (1,)), ((), ())),
                                 preferred_element_type=jnp.float32) + blin_ref[...]


def _tc_layer(aggp, cntp, x, Wl, Wr, b, Wlin=None, blin=None):
    final = Wlin is not None
    grid = (N // _RB,)
    in_specs = [
        pl.BlockSpec((NC, _RB, D), lambda i: (0, i, 0)),
        pl.BlockSpec((NC, _RB, CW), lambda i: (0, i, 0)),
        pl.BlockSpec((_RB, D), lambda i: (i, 0)),
        pl.BlockSpec((D, D), lambda i: (0, 0)),
        pl.BlockSpec((D, D), lambda i: (0, 0)),
        pl.BlockSpec((1, D), lambda i: (0, 0)),
    ]
    args = [aggp, cntp, x, Wl, Wr, b.reshape(1, D)]
    if final:
        in_specs += [pl.BlockSpec((D, D), lambda i: (0, 0)),
                     pl.BlockSpec((1, D), lambda i: (0, 0))]
        args += [Wlin, blin.reshape(1, D)]
    return pl.pallas_call(
        _tc_body_final if final else _tc_body,
        grid=grid,
        in_specs=in_specs,
        out_specs=pl.BlockSpec((_RB, D), lambda i: (i, 0)),
        out_shape=jax.ShapeDtypeStruct((N, D), jnp.float32),
    )(*args)


def kernel(x, edge_index, W1l, W1r, b1, W2l, W2r, b2, W3l, W3r, b3,
           Wlin, blin):
    src = edge_index[0].reshape(E // CHUNK, CHUNK)
    dst = edge_index[1].reshape(E // CHUNK, CHUNK)
    aggp, cntp = _sc_agg_cnt(x, src, dst)
    h1 = _tc_layer(aggp, cntp, x, W1l, W1r, b1)
    (aggp2,) = _sc_agg(h1, src, dst)
    h2 = _tc_layer(aggp2, cntp, h1, W2l, W2r, b2)
    (aggp3,) = _sc_agg(h2, src, dst)
    out = _tc_layer(aggp3, cntp, h2, W3l, W3r, b3, Wlin, blin)
    return out


# speedup vs baseline: 9.9947x; 1.1135x over previous
import functools

import jax
import jax.numpy as jnp
from jax import lax
from jax.experimental import pallas as pl
from jax.experimental.pallas import tpu as pltpu
from jax.experimental.pallas import tpu_sc as plsc

N = 10000
E = 320000
D = 128
NC = 2
NS = 16
NW = NC * NS
EPW = E // NW
CHUNK_A = 80
CHUNK_B = 125
RPT = 624
TAIL = N - RPT * NS
ZROWS = 24
CW = 16


def _fill2d(ref, nrows, ncols, value):
    vec = jnp.full((16,), value, jnp.float32)

    def row(r, _):
        def col(c, _2):
            ref[r, pl.ds(c * 16, 16)] = vec
            return 0
        return lax.fori_loop(0, ncols // 16, col, 0)
    lax.fori_loop(0, nrows, row, 0)


def _make_sc_agg(with_counts, CHUNK):
    NCHUNK = EPW // CHUNK
    mesh = plsc.VectorSubcoreMesh(core_axis_name="c", subcore_axis_name="s")
    out_type = [jax.ShapeDtypeStruct((NC, N, D), jnp.float32)]
    scratch = [
        pltpu.VMEM_SHARED((N, D), jnp.float32),
        pltpu.VMEM((NCHUNK, CHUNK), jnp.int32),
        pltpu.VMEM((CHUNK,), jnp.int32),
        pltpu.VMEM((CHUNK,), jnp.int32),
        pltpu.VMEM((CHUNK, D), jnp.float32),
        pltpu.VMEM((CHUNK, D), jnp.float32),
        pltpu.SemaphoreType.DMA,
        pltpu.SemaphoreType.DMA,
        pltpu.SemaphoreType.DMA,
        pltpu.SemaphoreType.DMA,
        pltpu.SemaphoreType.DMA,
        pltpu.SemaphoreType.DMA,
    ]
    if with_counts:
        out_type.append(jax.ShapeDtypeStruct((NC, N, CW), jnp.float32))
        scratch += [
            pltpu.VMEM_SHARED((N, CW), jnp.float32),
            pltpu.VMEM((CHUNK, CW), jnp.float32),
            pltpu.SemaphoreType.DMA,
            pltpu.SemaphoreType.DMA,
        ]

    @functools.partial(pl.kernel, mesh=mesh, out_type=tuple(out_type),
                       scratch_types=tuple(scratch),
                       compiler_params=pltpu.CompilerParams(
                           use_tc_tiling_on_sc=False))
    def k(x_hbm, src_hbm, dst_hbm, *refs):
        if with_counts:
            (agg_out, cnt_out, agg_sh, didx, six0, six1, rows0, rows1,
             sg0, sg1, ss0, ss1, si0, si1, cnt_sh, ones, sc0, sc1) = refs
        else:
            (agg_out, agg_sh, didx, six0, six1, rows0, rows1,
             sg0, sg1, ss0, ss1, si0, si1) = refs
        six = (six0, six1)
        rows = (rows0, rows1)
        sg = (sg0, sg1)
        ss = (ss0, ss1)
        si = (si0, si1)
        if with_counts:
            sc = (sc0, sc1)
        cid = lax.axis_index("c")
        sid = lax.axis_index("s")
        wid = cid * NS + sid
        is_last = sid == NS - 1
        brow = wid * NCHUNK

        pltpu.sync_copy(dst_hbm.at[pl.ds(brow, NCHUNK)], didx)

        _fill2d(rows0, CHUNK, D, 0.0)
        for j in range(RPT // CHUNK):
            pltpu.sync_copy(rows0, agg_sh.at[pl.ds(sid * RPT + j * CHUNK, CHUNK)])
        rem = RPT - (RPT // CHUNK) * CHUNK
        pltpu.sync_copy(rows0.at[pl.ds(0, rem)],
                        agg_sh.at[pl.ds(sid * RPT + RPT - rem, rem)])

        @pl.when(is_last)
        def _():
            pltpu.sync_copy(rows0.at[pl.ds(0, TAIL)],
                            agg_sh.at[pl.ds(NS * RPT, TAIL)])
        if with_counts:
            _fill2d(ones, CHUNK, CW, 0.0)
            for j in range(RPT // CHUNK):
                pltpu.sync_copy(ones, cnt_sh.at[pl.ds(sid * RPT + j * CHUNK, CHUNK)])
            pltpu.sync_copy(ones.at[pl.ds(0, rem)],
                            cnt_sh.at[pl.ds(sid * RPT + RPT - rem, rem)])

            @pl.when(is_last)
            def _():
                pltpu.sync_copy(ones.at[pl.ds(0, TAIL)],
                                cnt_sh.at[pl.ds(NS * RPT, TAIL)])
            _fill2d(ones, CHUNK, CW, 1.0)
        plsc.subcore_barrier()

        pltpu.sync_copy(src_hbm.at[brow], six[0])
        pltpu.async_copy(x_hbm.at[six[0]], rows[0], sg[0])
        pltpu.async_copy(src_hbm.at[brow + 1], six[1], si[1])

        def chunk(i, _):
            b = lax.rem(i, 2)

            def piece(bs):
                o = 1 - bs
                pltpu.make_async_copy(x_hbm.at[six[bs]], rows[bs], sg[bs]).wait()
                nxt = jnp.minimum(i + 2, NCHUNK - 1)
                pltpu.async_copy(src_hbm.at[brow + nxt], six[bs], si[bs])
                pltpu.async_copy(rows[bs], agg_sh.at[didx.at[i]], ss[bs],
                                 add=True)
                if with_counts:
                    pltpu.async_copy(ones, cnt_sh.at[didx.at[i]], sc[bs],
                                     add=True)

                @pl.when(i > 0)
                def _():
                    pltpu.make_async_copy(rows[o], agg_sh.at[didx.at[i]],
                                          ss[o]).wait()
                    if with_counts:
                        pltpu.make_async_copy(ones, cnt_sh.at[didx.at[i]],
                                              sc[o]).wait()
                pltpu.make_async_copy(src_hbm.at[brow], six[o], si[o]).wait()
                pltpu.async_copy(x_hbm.at[six[o]], rows[o], sg[o])

            lax.cond(b == 0, lambda: piece(0), lambda: piece(1))
            return 0
        lax.fori_loop(0, NCHUNK - 1, chunk, 0)

        lb = (NCHUNK - 1) % 2
        lo = 1 - lb
        pltpu.make_async_copy(x_hbm.at[six[lb]], rows[lb], sg[lb]).wait()
        pltpu.async_copy(rows[lb], agg_sh.at[didx.at[NCHUNK - 1]], ss[lb],
                         add=True)
        if with_counts:
            pltpu.async_copy(ones, cnt_sh.at[didx.at[NCHUNK - 1]], sc[lb],
                             add=True)
        pltpu.make_async_copy(rows[lb], agg_sh.at[didx.at[0]], ss[lb]).wait()
        pltpu.make_async_copy(rows[lo], agg_sh.at[didx.at[0]], ss[lo]).wait()
        pltpu.make_async_copy(src_hbm.at[brow], six[lo], si[lo]).wait()
        if with_counts:
            pltpu.make_async_copy(ones, cnt_sh.at[didx.at[0]], sc[lb]).wait()
            pltpu.make_async_copy(ones, cnt_sh.at[didx.at[0]], sc[lo]).wait()

        plsc.subcore_barrier()

        r0 = sid * RPT
        pltpu.sync_copy(agg_sh.at[pl.ds(r0, RPT)],
                        agg_out.at[cid, pl.ds(r0, RPT)])

        @pl.when(is_last)
        def _():
            pltpu.sync_copy(agg_sh.at[pl.ds(NS * RPT, TAIL)],
                            agg_out.at[cid, pl.ds(NS * RPT, TAIL)])
        if with_counts:
            pltpu.sync_copy(cnt_sh.at[pl.ds(r0, RPT)],
                            cnt_out.at[cid, pl.ds(r0, RPT)])

            @pl.when(is_last)
            def _():
                pltpu.sync_copy(cnt_sh.at[pl.ds(NS * RPT, TAIL)],
                                cnt_out.at[cid, pl.ds(NS * RPT, TAIL)])

    return k


_sc_agg_cnt = _make_sc_agg(True, CHUNK_A)
_sc_agg = _make_sc_agg(False, CHUNK_B)

_RB = 1000


def _tc_body(agg_ref, cnt_ref, x_ref, wl_ref, wr_ref, b_ref, o_ref):
    agg = agg_ref[0] + agg_ref[1]
    cnt = cnt_ref[0, :, 0:1] + cnt_ref[1, :, 0:1]
    m = agg / jnp.maximum(cnt, 1.0)
    h = lax.dot_general(m, wl_ref[...], (((1,), (1,)), ((), ())),
                        preferred_element_type=jnp.float32)
    h += lax.dot_general(x_ref[...], wr_ref[...], (((1,), (1,)), ((), ())),
                         preferred_element_type=jnp.float32)
    h += b_ref[...]
    o_ref[...] = jnp.maximum(h, 0.0)


def _tc_body_final(agg_ref, cnt_ref, x_ref, wl_ref, wr_ref, b_ref,
                   wlin_ref, blin_ref, o_ref):
    agg = agg_ref[0] + agg_ref[1]
    cnt = cnt_ref[0, :, 0:1] + cnt_ref[1, :, 0:1]
    m = agg / jnp.maximum(cnt, 1.0)
    h = lax.dot_general(m, wl_ref[...], (((1,), (1,)), ((), ())),
                        preferred_element_type=jnp.float32)
    h += lax.dot_general(x_ref[...], wr_ref[...], (((1,), (1,)), ((), ())),
                         preferred_element_type=jnp.float32)
    h += b_ref[...]
    h = jnp.maximum(h, 0.0)
    o_ref[...] = lax.dot_general(h, wlin_ref[...], (((1,), (1,)), ((), ())),
                                 preferred_element_type=jnp.float32) + blin_ref[...]


def _tc_layer(aggp, cntp, x, Wl, Wr, b, Wlin=None, blin=None):
    final = Wlin is not None
    grid = (N // _RB,)
    in_specs = [
        pl.BlockSpec((NC, _RB, D), lambda i: (0, i, 0)),
        pl.BlockSpec((NC, _RB, CW), lambda i: (0, i, 0)),
        pl.BlockSpec((_RB, D), lambda i: (i, 0)),
        pl.BlockSpec((D, D), lambda i: (0, 0)),
        pl.BlockSpec((D, D), lambda i: (0, 0)),
        pl.BlockSpec((1, D), lambda i: (0, 0)),
    ]
    args = [aggp, cntp, x, Wl, Wr, b.reshape(1, D)]
    if final:
        in_specs += [pl.BlockSpec((D, D), lambda i: (0, 0)),
                     pl.BlockSpec((1, D), lambda i: (0, 0))]
        args += [Wlin, blin.reshape(1, D)]
    return pl.pallas_call(
        _tc_body_final if final else _tc_body,
        grid=grid,
        in_specs=in_specs,
        out_specs=pl.BlockSpec((_RB, D), lambda i: (i, 0)),
        out_shape=jax.ShapeDtypeStruct((N, D), jnp.float32),
    )(*args)


def kernel(x, edge_index, W1l, W1r, b1, W2l, W2r, b2, W3l, W3r, b3,
           Wlin, blin):
    src_a = edge_index[0].reshape(E // CHUNK_A, CHUNK_A)
    dst_a = edge_index[1].reshape(E // CHUNK_A, CHUNK_A)
    src_b = edge_index[0].reshape(E // CHUNK_B, CHUNK_B)
    dst_b = edge_index[1].reshape(E // CHUNK_B, CHUNK_B)
    aggp, cntp = _sc_agg_cnt(x, src_a, dst_a)
    h1 = _tc_layer(aggp, cntp, x, W1l, W1r, b1)
    (aggp2,) = _sc_agg(h1, src_b, dst_b)
    h2 = _tc_layer(aggp2, cntp, h1, W2l, W2r, b2)
    (aggp3,) = _sc_agg(h2, src_b, dst_b)
    out = _tc_layer(aggp3, cntp, h2, W3l, W3r, b3, Wlin, blin)
    return out
